# baseline (speedup 1.0000x reference)
import functools

import jax
import jax.numpy as jnp
from jax import lax
from jax.experimental import pallas as pl
from jax.experimental.pallas import tpu as pltpu
from jax.experimental.pallas import tpu_sc as plsc

_NUM_POS = 128
_NUM_NEG = 128
_LANES = 128
_ROWS = 160


def _select_kernel(tb_ref, planes_ref, pos_ref, neg_ref, ptgt_ref,
                   pkey_ref, nkey_ref, targ_ref,
                   *, n_valid, n_tgt):
    xc = planes_ref[0]
    yc = planes_ref[1]
    w = planes_ref[2]
    h = planes_ref[3]
    x0 = xc - w / 2
    y0 = yc - h / 2
    x1 = xc + w / 2
    y1 = yc + h / 2
    area_p = (x1 - x0) * (y1 - y0)

    def tgt_body(t, carry):
        miou, targ = carry
        txc = tb_ref[t, 0]
        tyc = tb_ref[t, 1]
        tw = tb_ref[t, 2]
        th = tb_ref[t, 3]
        tx0 = txc - tw / 2
        ty0 = tyc - th / 2
        tx1 = txc + tw / 2
        ty1 = tyc + th / 2
        area_t = (tx1 - tx0) * (ty1 - ty0)
        iw = jnp.maximum(jnp.minimum(x1, tx1) - jnp.maximum(x0, tx0), 0.0)
        ih = jnp.maximum(jnp.minimum(y1, ty1) - jnp.maximum(y0, ty0), 0.0)
        inter = iw * ih
        union = (area_p + area_t) - inter
        iou = inter / jnp.maximum(union, 1e-8)
        upd = iou > miou
        return jnp.where(upd, iou, miou), jnp.where(upd, t, targ)

    miou0 = jnp.full((_ROWS, _LANES), -jnp.inf, dtype=jnp.float32)
    targ0 = jnp.zeros((_ROWS, _LANES), dtype=jnp.int32)
    miou, targ = lax.fori_loop(0, n_tgt, tgt_body, (miou0, targ0))
    targ_ref[...] = targ

    lane = lax.broadcasted_iota(jnp.int32, (1, _LANES), 1)
    row = lax.broadcasted_iota(jnp.int32, (_ROWS, 1), 0)
    gidx = lane * _ROWS + row
    valid = gidx < n_valid
    ninf = jnp.float32(-jnp.inf)
    big = jnp.int32(2**20)

    pkey_ref[...] = jnp.where(valid, miou, ninf)
    nkey_ref[...] = jnp.where(valid, -miou, ninf)

    def col_best(key):
        mx = jnp.max(key, axis=0, keepdims=True)
        rw = jnp.min(jnp.where(key == mx, row, big), axis=0, keepdims=True)
        return mx, rw

    pcmax, pcrow = col_best(pkey_ref[...])
    ncmax, ncrow = col_best(nkey_ref[...])

    def extract(key_ref, cmax, crow):
        m = jnp.max(cmax, axis=1, keepdims=True)
        packed = jnp.where(cmax == m, lane * 1024 + crow, big)
        p = jnp.min(packed, axis=1, keepdims=True)
        c = p // 1024
        r = p % 1024
        lanec = lane == c
        hit = lanec & (row == r)
        key = jnp.where(hit, ninf, key_ref[...])
        key_ref[...] = key
        colvals = jnp.where(lanec, key, ninf)
        mx = jnp.max(colvals, axis=0, keepdims=True)
        rw = jnp.min(jnp.where(colvals == mx, row, big), axis=0, keepdims=True)
        cmax = jnp.where(lanec, mx, cmax)
        crow = jnp.where(lanec, rw, crow)
        return cmax, crow, c * _ROWS + r, hit

    def ext_body(i, s):
        pcmax, pcrow, ncmax, ncrow, pvec, nvec, tvec = s
        pcmax, pcrow, porig, phit = extract(pkey_ref, pcmax, pcrow)
        ncmax, ncrow, norig, _ = extract(nkey_ref, ncmax, ncrow)
        ptgt = jnp.max(jnp.max(jnp.where(phit, targ_ref[...], -1), axis=0,
                               keepdims=True), axis=1, keepdims=True)
        sel = lane == i
        pvec = jnp.where(sel, porig, pvec)
        nvec = jnp.where(sel, norig, nvec)
        tvec = jnp.where(sel, ptgt, tvec)
        return pcmax, pcrow, ncmax, ncrow, pvec, nvec, tvec

    zero = jnp.zeros((1, _LANES), dtype=jnp.int32)
    s = lax.fori_loop(0, _NUM_POS, ext_body,
                      (pcmax, pcrow, ncmax, ncrow, zero, zero, zero))
    pos_ref[...] = s[4]
    neg_ref[...] = s[5]
    ptgt_ref[...] = s[6]


def _select_indices(input_boxes, target_boxes):
    b1 = input_boxes.shape[1]
    n_tgt = target_boxes.shape[1]
    npad = _ROWS * _LANES
    planes = jnp.transpose(input_boxes[0])
    planes = jnp.pad(planes, ((0, 0), (0, npad - b1)))
    planes = planes.reshape(4, _LANES, _ROWS).transpose(0, 2, 1)
    idx_shape = jax.ShapeDtypeStruct((1, _LANES), jnp.int32)
    pos, neg, ptgt = pl.pallas_call(
        functools.partial(_select_kernel, n_valid=b1, n_tgt=n_tgt),
        out_shape=[idx_shape, idx_shape, idx_shape],
        in_specs=[
            pl.BlockSpec(memory_space=pltpu.SMEM),
            pl.BlockSpec(memory_space=pltpu.VMEM),
        ],
        out_specs=[pl.BlockSpec(memory_space=pltpu.VMEM)] * 3,
        scratch_shapes=[
            pltpu.VMEM((_ROWS, _LANES), jnp.float32),
            pltpu.VMEM((_ROWS, _LANES), jnp.float32),
            pltpu.VMEM((_ROWS, _LANES), jnp.int32),
        ],
    )(target_boxes[0], planes)
    return pos.reshape(-1), neg.reshape(-1), ptgt.reshape(-1)


def _gather_body(pos_hbm, neg_hbm, ptgt_hbm, ftab_hbm, ttab_hbm,
                 out_p, out_n, out_t,
                 idx_v, rows_v, tidx_v, trows_v, sem):
    wid = lax.axis_index("s") * 2 + lax.axis_index("c")
    is_pos = wid < 16
    base = jnp.where(is_pos, wid, wid - 16) * 8

    @pl.when(is_pos)
    def _():
        pltpu.sync_copy(pos_hbm.at[pl.ds(base, 8)], idx_v)
        pltpu.async_copy(ftab_hbm.at[idx_v], rows_v, sem).wait()
        pltpu.sync_copy(rows_v, out_p.at[pl.ds(base, 8)])
        pltpu.sync_copy(ptgt_hbm.at[pl.ds(base, 8)], tidx_v)
        pltpu.async_copy(ttab_hbm.at[tidx_v], trows_v, sem).wait()
        pltpu.sync_copy(trows_v, out_t.at[pl.ds(base, 8)])

    @pl.when(jnp.logical_not(is_pos))
    def _():
        pltpu.sync_copy(neg_hbm.at[pl.ds(base, 8)], idx_v)
        pltpu.async_copy(ftab_hbm.at[idx_v], rows_v, sem).wait()
        pltpu.sync_copy(rows_v, out_n.at[pl.ds(base, 8)])


def _gather_sc(pos_idx, neg_idx, ptgt_idx, ftable, ttable):
    f32 = jnp.float32
    i32 = jnp.int32
    fw = ftable.shape[1]
    tw = ttable.shape[1]
    run = pl.kernel(
        _gather_body,
        out_type=[
            jax.ShapeDtypeStruct((_NUM_POS, fw), f32),
            jax.ShapeDtypeStruct((_NUM_NEG, fw), f32),
            jax.ShapeDtypeStruct((_NUM_POS, tw), i32),
        ],
        mesh=plsc.VectorSubcoreMesh(core_axis_name="c", subcore_axis_name="s"),
        compiler_params=pltpu.CompilerParams(use_tc_tiling_on_sc=False),
        scratch_types=[
            pltpu.VMEM((8,), i32),
            pltpu.VMEM((8, fw), f32),
            pltpu.VMEM((8,), i32),
            pltpu.VMEM((8, tw), i32),
            pltpu.SemaphoreType.DMA,
        ],
    )
    return run(pos_idx, neg_idx, ptgt_idx, ftable, ttable)


def kernel(input_boxes, input_anchors, input_trans, input_scores,
           target_boxes, target_labels):
    pos_idx, neg_idx, ptgt_idx = _select_indices(input_boxes, target_boxes)

    b1 = input_boxes.shape[1]
    b2 = target_boxes.shape[1]
    nlab = target_labels.shape[2]
    ftable = jnp.concatenate(
        [input_boxes[0], input_anchors[0], input_trans[0], input_scores[0],
         jnp.zeros((b1, 3), jnp.float32)], axis=1)
    ttable = jnp.concatenate(
        [lax.bitcast_convert_type(target_boxes[0], jnp.int32),
         target_labels[0],
         jnp.zeros((b2, 32 - 4 - nlab), jnp.int32)], axis=1)
    prow, nrow, trow = _gather_sc(pos_idx, neg_idx, ptgt_idx, ftable, ttable)
    return (
        prow[:, 0:4], nrow[:, 0:4],
        prow[:, 4:8], nrow[:, 4:8],
        prow[:, 8:12], nrow[:, 8:12],
        prow[:, 12:13], nrow[:, 12:13],
        lax.bitcast_convert_type(trow[:, 0:4], jnp.float32),
        trow[:, 4:4 + nlab],
    )
